# baseline (speedup 1.0000x reference)
import functools

import jax
import jax.numpy as jnp
from jax import lax
from jax.experimental import pallas as pl
from jax.experimental.pallas import tpu as pltpu
from jax.experimental.pallas import tpu_sc as plsc


def _cdiv(a, b):
    return (a + b - 1) // b


def _rowsums_scaled(table, scale):
    n, d = table.shape
    blk = 1024
    nrows = _cdiv(n, 128)
    ngrid = _cdiv(n, blk)

    def body(t_ref, o_ref):
        x = t_ref[...].reshape(blk // 128, 128, d)
        o_ref[...] = jnp.sum(x, axis=2) * scale

    out = pl.pallas_call(
        body,
        grid=(ngrid,),
        in_specs=[pl.BlockSpec((blk, d), lambda i: (i, 0))],
        out_specs=pl.BlockSpec((blk // 128, 128), lambda i: (i, 0)),
        out_shape=jax.ShapeDtypeStruct((nrows, 128), jnp.float32),
    )(table)
    return out.reshape(-1)


def _sc_gather_sum(rs_w, rs_2, rs_3, idx_w, idx_2, idx_3):
    info = plsc.get_sparse_core_info()
    nw = info.num_cores * info.num_subcores
    ntot = idx_w.shape[0]
    per_w = ntot // nw
    chunk = 8192
    nchunk = per_w // chunk
    assert ntot % nw == 0 and per_w % chunk == 0
    mesh = plsc.VectorSubcoreMesh(core_axis_name="c", subcore_axis_name="s")

    vmem_i = pltpu.VMEM((chunk,), jnp.int32)
    vmem_f = pltpu.VMEM((chunk,), jnp.float32)

    @functools.partial(
        pl.kernel,
        mesh=mesh,
        out_type=jax.ShapeDtypeStruct((ntot,), jnp.float32),
        scratch_types=[
            [[vmem_i] * 3] * 2,
            [[vmem_f] * 3] * 2,
            [vmem_f] * 2,
            [[pltpu.SemaphoreType.DMA] * 3] * 2,
            [pltpu.SemaphoreType.DMA] * 2,
        ],
    )
    def k(rsw_h, rs2_h, rs3_h, iw_h, i2_h, i3_h, out_h,
          idx_v, val_v, obuf_v, gsem, osem):
        wid = lax.axis_index("s") * info.num_cores + lax.axis_index("c")
        rs_h = (rsw_h, rs2_h, rs3_h)
        ix_h = (iw_h, i2_h, i3_h)
        gathers = [[None] * 3 for _ in range(nchunk)]
        stores = [None] * nchunk
        for c in range(nchunk + 1):
            if c < nchunk:
                b = c % 2
                base = wid * per_w + c * chunk
                for t in range(3):
                    pltpu.sync_copy(ix_h[t].at[pl.ds(base, chunk)], idx_v[b][t])
                    gathers[c][t] = pltpu.async_copy(
                        rs_h[t].at[idx_v[b][t]], val_v[b][t], gsem[b][t])
            if c >= 1:
                d = c - 1
                pb = d % 2
                for t in range(3):
                    gathers[d][t].wait()
                if d >= 2:
                    stores[d - 2].wait()

                def add_vec(i, _, pb=pb):
                    s = pl.ds(i * 16, 16)
                    obuf_v[pb][s] = (val_v[pb][0][s] + val_v[pb][1][s]
                                     + val_v[pb][2][s])
                    return 0

                lax.fori_loop(0, chunk // 16, add_vec, 0, unroll=8)
                dbase = wid * per_w + d * chunk
                stores[d] = pltpu.async_copy(
                    obuf_v[pb], out_h.at[pl.ds(dbase, chunk)], osem[pb])
        stores[nchunk - 2].wait()
        stores[nchunk - 1].wait()

    return k(rs_w, rs_2, rs_3, idx_w, idx_2, idx_3)


def _mlp_head(x, w1, b1, w2p, b2p):
    bsz, l = x.shape
    d = w1.shape[1]
    blk = 512

    def body(x_ref, w1_ref, b1_ref, w2_ref, b2_ref, o_ref):
        h = jnp.dot(x_ref[...], w1_ref[...], preferred_element_type=jnp.float32)
        h = jnp.maximum(h + b1_ref[...], 0.0)
        o_ref[...] = (
            jnp.dot(h, w2_ref[...], preferred_element_type=jnp.float32)
            + b2_ref[...]
        )

    return pl.pallas_call(
        body,
        grid=(bsz // blk,),
        in_specs=[
            pl.BlockSpec((blk, l), lambda i: (i, 0)),
            pl.BlockSpec((l, d), lambda i: (0, 0)),
            pl.BlockSpec((1, d), lambda i: (0, 0)),
            pl.BlockSpec((d, d), lambda i: (0, 0)),
            pl.BlockSpec((1, d), lambda i: (0, 0)),
        ],
        out_specs=pl.BlockSpec((blk, d), lambda i: (i, 0)),
        out_shape=jax.ShapeDtypeStruct((bsz, d), jnp.float32),
    )(x, w1, b1, w2p, b2p)


def kernel(input_ids, input_ids_gram2, input_ids_gram3, input_mask, labels,
           emb_word, emb_g2, emb_g3, W1, b1, W2, b2):
    bsz, l = input_ids.shape
    d = W1.shape[1]
    num_labels = W2.shape[1]
    scale = 1.0 / float(l)

    rs_w = _rowsums_scaled(emb_word, scale)
    rs_2 = _rowsums_scaled(emb_g2, scale)
    rs_3 = _rowsums_scaled(emb_g3, scale)

    pooled = _sc_gather_sum(
        rs_w, rs_2, rs_3,
        input_ids.reshape(-1),
        input_ids_gram2.reshape(-1),
        input_ids_gram3.reshape(-1),
    ).reshape(bsz, l)

    w2p = jnp.zeros((d, d), jnp.float32).at[:, :num_labels].set(W2)
    b2p = jnp.zeros((1, d), jnp.float32).at[0, :num_labels].set(b2)
    out_full = _mlp_head(pooled, W1, b1.reshape(1, d), w2p, b2p)
    return out_full[:, :num_labels]


# speedup vs baseline: 2.1697x; 2.1697x over previous
import functools

import jax
import jax.numpy as jnp
from jax import lax
from jax.experimental import pallas as pl
from jax.experimental.pallas import tpu as pltpu
from jax.experimental.pallas import tpu_sc as plsc


def _cdiv(a, b):
    return (a + b - 1) // b


def _rowsums_scaled(table, scale):
    n, d = table.shape
    blk = 4096
    nrows = _cdiv(n, 128)
    ngrid = _cdiv(n, blk)

    def body(t_ref, o_ref):
        x = t_ref[...].reshape(blk // 128, 128, d)
        o_ref[...] = jnp.sum(x, axis=2) * scale

    out = pl.pallas_call(
        body,
        grid=(ngrid,),
        in_specs=[pl.BlockSpec((blk, d), lambda i: (i, 0))],
        out_specs=pl.BlockSpec((blk // 128, 128), lambda i: (i, 0)),
        out_shape=jax.ShapeDtypeStruct((nrows, 128), jnp.float32),
    )(table)
    return out.reshape(-1)


def _sc_gather_sum(rs_w, rs_2, rs_3, idx_w, idx_2, idx_3):
    info = plsc.get_sparse_core_info()
    nw = info.num_cores * info.num_subcores
    ns = info.num_subcores
    ntot = idx_w.shape[0]
    per_w = ntot // nw
    chunk = 4096
    nchunk = per_w // chunk
    assert ntot % nw == 0 and per_w % chunk == 0
    sizes = (rs_w.shape[0], rs_2.shape[0], rs_3.shape[0])
    assert all(sz % (8 * ns) == 0 for sz in sizes)
    mesh = plsc.VectorSubcoreMesh(core_axis_name="c", subcore_axis_name="s")

    vmem_i = pltpu.VMEM((chunk,), jnp.int32)
    vmem_f = pltpu.VMEM((chunk,), jnp.float32)

    @functools.partial(
        pl.kernel,
        mesh=mesh,
        out_type=jax.ShapeDtypeStruct((ntot,), jnp.float32),
        scratch_types=[
            [pltpu.VMEM_SHARED((sz,), jnp.float32) for sz in sizes],
            [[vmem_i] * 3] * 2,
            [[vmem_f] * 3] * 2,
            [vmem_f] * 2,
            [[pltpu.SemaphoreType.DMA] * 3] * 2,
            [pltpu.SemaphoreType.DMA] * 2,
        ],
    )
    def k(rsw_h, rs2_h, rs3_h, iw_h, i2_h, i3_h, out_h,
          rs_s, idx_v, val_v, obuf_v, gsem, osem):
        wid = lax.axis_index("s") * info.num_cores + lax.axis_index("c")
        sid = lax.axis_index("s")
        for t, src in enumerate((rsw_h, rs2_h, rs3_h)):
            sl = sizes[t] // ns
            done = 0
            while done < sl:
                piece = min(chunk, sl - done)
                off = sid * sl + done
                pltpu.sync_copy(src.at[pl.ds(off, piece)],
                                obuf_v[0].at[pl.ds(0, piece)])
                pltpu.sync_copy(obuf_v[0].at[pl.ds(0, piece)],
                                rs_s[t].at[pl.ds(off, piece)])
                done += piece
        plsc.subcore_barrier()
        rs_h = (rs_s[0], rs_s[1], rs_s[2])
        ix_h = (iw_h, i2_h, i3_h)
        gathers = [[None] * 3 for _ in range(nchunk)]
        stores = [None] * nchunk
        for c in range(nchunk + 1):
            if c < nchunk:
                b = c % 2
                base = wid * per_w + c * chunk
                for t in range(3):
                    pltpu.sync_copy(ix_h[t].at[pl.ds(base, chunk)], idx_v[b][t])
                    gathers[c][t] = pltpu.async_copy(
                        rs_h[t].at[idx_v[b][t]], val_v[b][t], gsem[b][t])
            if c >= 1:
                d = c - 1
                pb = d % 2
                for t in range(3):
                    gathers[d][t].wait()
                if d >= 2:
                    stores[d - 2].wait()

                def add_vec(i, _, pb=pb):
                    s = pl.ds(i * 16, 16)
                    obuf_v[pb][s] = (val_v[pb][0][s] + val_v[pb][1][s]
                                     + val_v[pb][2][s])
                    return 0

                lax.fori_loop(0, chunk // 16, add_vec, 0, unroll=8)
                dbase = wid * per_w + d * chunk
                stores[d] = pltpu.async_copy(
                    obuf_v[pb], out_h.at[pl.ds(dbase, chunk)], osem[pb])
        stores[nchunk - 2].wait()
        stores[nchunk - 1].wait()

    return k(rs_w, rs_2, rs_3, idx_w, idx_2, idx_3)


def _mlp_head(x, w1, b1, w2p, b2p):
    bsz, l = x.shape
    d = w1.shape[1]
    blk = 512

    def body(x_ref, w1_ref, b1_ref, w2_ref, b2_ref, o_ref):
        h = jnp.dot(x_ref[...], w1_ref[...], preferred_element_type=jnp.float32)
        h = jnp.maximum(h + b1_ref[...], 0.0)
        o_ref[...] = (
            jnp.dot(h, w2_ref[...], preferred_element_type=jnp.float32)
            + b2_ref[...]
        )

    return pl.pallas_call(
        body,
        grid=(bsz // blk,),
        in_specs=[
            pl.BlockSpec((blk, l), lambda i: (i, 0)),
            pl.BlockSpec((l, d), lambda i: (0, 0)),
            pl.BlockSpec((1, d), lambda i: (0, 0)),
            pl.BlockSpec((d, d), lambda i: (0, 0)),
            pl.BlockSpec((1, d), lambda i: (0, 0)),
        ],
        out_specs=pl.BlockSpec((blk, d), lambda i: (i, 0)),
        out_shape=jax.ShapeDtypeStruct((bsz, d), jnp.float32),
    )(x, w1, b1, w2p, b2p)


def kernel(input_ids, input_ids_gram2, input_ids_gram3, input_mask, labels,
           emb_word, emb_g2, emb_g3, W1, b1, W2, b2):
    bsz, l = input_ids.shape
    d = W1.shape[1]
    num_labels = W2.shape[1]
    scale = 1.0 / float(l)

    rs_w = _rowsums_scaled(emb_word, scale)
    rs_2 = _rowsums_scaled(emb_g2, scale)
    rs_3 = _rowsums_scaled(emb_g3, scale)

    pooled = _sc_gather_sum(
        rs_w, rs_2, rs_3,
        input_ids.reshape(-1),
        input_ids_gram2.reshape(-1),
        input_ids_gram3.reshape(-1),
    ).reshape(bsz, l)

    w2p = jnp.zeros((d, d), jnp.float32).at[:, :num_labels].set(W2)
    b2p = jnp.zeros((1, d), jnp.float32).at[0, :num_labels].set(b2)
    out_full = _mlp_head(pooled, W1, b1.reshape(1, d), w2p, b2p)
    return out_full[:, :num_labels]


# speedup vs baseline: 2.5230x; 1.1628x over previous
import functools

import jax
import jax.numpy as jnp
from jax import lax
from jax.experimental import pallas as pl
from jax.experimental.pallas import tpu as pltpu
from jax.experimental.pallas import tpu_sc as plsc


def _cdiv(a, b):
    return (a + b - 1) // b


def _rowsums_scaled(table, scale):
    n, d = table.shape
    blk = 8192
    nrows = _cdiv(n, 128)
    ngrid = _cdiv(n, blk)

    def body(t_ref, o_ref):
        x = t_ref[...].reshape(blk // 128, 128, d)
        o_ref[...] = jnp.sum(x, axis=2) * scale

    out = pl.pallas_call(
        body,
        grid=(ngrid,),
        in_specs=[pl.BlockSpec((blk, d), lambda i: (i, 0))],
        out_specs=pl.BlockSpec((blk // 128, 128), lambda i: (i, 0)),
        out_shape=jax.ShapeDtypeStruct((nrows, 128), jnp.float32),
    )(table)
    return out.reshape(-1)


def _sc_gather_sum(rs_w, rs_2, rs_3, idx_w, idx_2, idx_3):
    info = plsc.get_sparse_core_info()
    nw = info.num_cores * info.num_subcores
    ns = info.num_subcores
    ntot = idx_w.shape[0]
    per_w = ntot // nw
    chunk = 4096
    nchunk = per_w // chunk
    assert ntot % nw == 0 and per_w % chunk == 0
    sizes = (rs_w.shape[0], rs_2.shape[0], rs_3.shape[0])
    assert all(sz % (8 * ns) == 0 for sz in sizes)
    mesh = plsc.VectorSubcoreMesh(core_axis_name="c", subcore_axis_name="s")

    vmem_i = pltpu.VMEM((chunk,), jnp.int32)
    vmem_f = pltpu.VMEM((chunk,), jnp.float32)

    @functools.partial(
        pl.kernel,
        mesh=mesh,
        out_type=jax.ShapeDtypeStruct((ntot,), jnp.float32),
        scratch_types=[
            [pltpu.VMEM_SHARED((sz,), jnp.float32) for sz in sizes],
            [[vmem_i] * 3] * 2,
            [[vmem_f] * 3] * 2,
            [vmem_f] * 2,
            [[pltpu.SemaphoreType.DMA] * 3] * 2,
            [pltpu.SemaphoreType.DMA] * 2,
        ],
    )
    def k(rsw_h, rs2_h, rs3_h, iw_h, i2_h, i3_h, out_h,
          rs_s, idx_v, val_v, obuf_v, gsem, osem):
        wid = lax.axis_index("s") * info.num_cores + lax.axis_index("c")
        sid = lax.axis_index("s")
        for t, src in enumerate((rsw_h, rs2_h, rs3_h)):
            sl = sizes[t] // ns
            done = 0
            while done < sl:
                piece = min(chunk, sl - done)
                off = sid * sl + done
                pltpu.sync_copy(src.at[pl.ds(off, piece)],
                                obuf_v[0].at[pl.ds(0, piece)])
                pltpu.sync_copy(obuf_v[0].at[pl.ds(0, piece)],
                                rs_s[t].at[pl.ds(off, piece)])
                done += piece
        plsc.subcore_barrier()
        rs_h = (rs_s[0], rs_s[1], rs_s[2])
        ix_h = (iw_h, i2_h, i3_h)
        gathers = [[None] * 3 for _ in range(nchunk)]
        stores = [None] * nchunk
        for c in range(nchunk + 1):
            if c < nchunk:
                b = c % 2
                base = wid * per_w + c * chunk
                for t in range(3):
                    pltpu.sync_copy(ix_h[t].at[pl.ds(base, chunk)], idx_v[b][t])
                    gathers[c][t] = pltpu.async_copy(
                        rs_h[t].at[idx_v[b][t]], val_v[b][t], gsem[b][t])
            if c >= 1:
                d = c - 1
                pb = d % 2
                for t in range(3):
                    gathers[d][t].wait()
                if d >= 2:
                    stores[d - 2].wait()

                def add_vec(i, _, pb=pb):
                    s = pl.ds(i * 16, 16)
                    obuf_v[pb][s] = (val_v[pb][0][s] + val_v[pb][1][s]
                                     + val_v[pb][2][s])
                    return 0

                lax.fori_loop(0, chunk // 16, add_vec, 0, unroll=8)
                dbase = wid * per_w + d * chunk
                stores[d] = pltpu.async_copy(
                    obuf_v[pb], out_h.at[pl.ds(dbase, chunk)], osem[pb])
        stores[nchunk - 2].wait()
        stores[nchunk - 1].wait()

    return k(rs_w, rs_2, rs_3, idx_w, idx_2, idx_3)


def _mlp_head(x, w1, b1, w2p, b2p):
    bsz, l = x.shape
    d = w1.shape[1]
    blk = 512

    def body(x_ref, w1_ref, b1_ref, w2_ref, b2_ref, o_ref):
        h = jnp.dot(x_ref[...], w1_ref[...], preferred_element_type=jnp.float32)
        h = jnp.maximum(h + b1_ref[...], 0.0)
        o_ref[...] = (
            jnp.dot(h, w2_ref[...], preferred_element_type=jnp.float32)
            + b2_ref[...]
        )

    return pl.pallas_call(
        body,
        grid=(bsz // blk,),
        in_specs=[
            pl.BlockSpec((blk, l), lambda i: (i, 0)),
            pl.BlockSpec((l, d), lambda i: (0, 0)),
            pl.BlockSpec((1, d), lambda i: (0, 0)),
            pl.BlockSpec((d, d), lambda i: (0, 0)),
            pl.BlockSpec((1, d), lambda i: (0, 0)),
        ],
        out_specs=pl.BlockSpec((blk, d), lambda i: (i, 0)),
        out_shape=jax.ShapeDtypeStruct((bsz, d), jnp.float32),
    )(x, w1, b1, w2p, b2p)


def kernel(input_ids, input_ids_gram2, input_ids_gram3, input_mask, labels,
           emb_word, emb_g2, emb_g3, W1, b1, W2, b2):
    bsz, l = input_ids.shape
    d = W1.shape[1]
    num_labels = W2.shape[1]
    scale = 1.0 / float(l)

    rs_w = _rowsums_scaled(emb_word, scale)
    rs_2 = _rowsums_scaled(emb_g2, scale)
    rs_3 = _rowsums_scaled(emb_g3, scale)

    pooled = _sc_gather_sum(
        rs_w, rs_2, rs_3,
        input_ids.reshape(-1),
        input_ids_gram2.reshape(-1),
        input_ids_gram3.reshape(-1),
    ).reshape(bsz, l)

    w2p = jnp.zeros((d, d), jnp.float32).at[:, :num_labels].set(W2)
    b2p = jnp.zeros((1, d), jnp.float32).at[0, :num_labels].set(b2)
    out_full = _mlp_head(pooled, W1, b1.reshape(1, d), w2p, b2p)
    return out_full[:, :num_labels]


# speedup vs baseline: 2.6785x; 1.0616x over previous
import functools

import jax
import jax.numpy as jnp
from jax import lax
from jax.experimental import pallas as pl
from jax.experimental.pallas import tpu as pltpu
from jax.experimental.pallas import tpu_sc as plsc


def _cdiv(a, b):
    return (a + b - 1) // b


def _rowsums_scaled(table, scale):
    n, d = table.shape
    blk = 16384
    nrows = _cdiv(n, 128)
    ngrid = _cdiv(n, blk)

    def body(t_ref, o_ref):
        x = t_ref[...].reshape(blk // 128, 128, d)
        o_ref[...] = jnp.sum(x, axis=2) * scale

    out = pl.pallas_call(
        body,
        grid=(ngrid,),
        in_specs=[pl.BlockSpec((blk, d), lambda i: (i, 0))],
        out_specs=pl.BlockSpec((blk // 128, 128), lambda i: (i, 0)),
        out_shape=jax.ShapeDtypeStruct((nrows, 128), jnp.float32),
    )(table)
    return out.reshape(-1)


def _sc_gather_sum(rs_w, rs_2, rs_3, idx_w, idx_2, idx_3):
    info = plsc.get_sparse_core_info()
    nw = info.num_cores * info.num_subcores
    ns = info.num_subcores
    ntot = idx_w.shape[0]
    per_w = ntot // nw
    chunk = 4096
    nchunk = per_w // chunk
    assert ntot % nw == 0 and per_w % chunk == 0
    sizes = (rs_w.shape[0], rs_2.shape[0], rs_3.shape[0])
    assert all(sz % (8 * ns) == 0 for sz in sizes)
    mesh = plsc.VectorSubcoreMesh(core_axis_name="c", subcore_axis_name="s")

    vmem_i = pltpu.VMEM((chunk,), jnp.int32)
    vmem_f = pltpu.VMEM((chunk,), jnp.float32)

    @functools.partial(
        pl.kernel,
        mesh=mesh,
        out_type=jax.ShapeDtypeStruct((ntot,), jnp.float32),
        scratch_types=[
            [pltpu.VMEM_SHARED((sz,), jnp.float32) for sz in sizes],
            [[vmem_i] * 3] * 2,
            [[vmem_f] * 3] * 2,
            [vmem_f] * 2,
            [[pltpu.SemaphoreType.DMA] * 3] * 2,
            [pltpu.SemaphoreType.DMA] * 2,
        ],
    )
    def k(rsw_h, rs2_h, rs3_h, iw_h, i2_h, i3_h, out_h,
          rs_s, idx_v, val_v, obuf_v, gsem, osem):
        wid = lax.axis_index("s") * info.num_cores + lax.axis_index("c")
        sid = lax.axis_index("s")
        for t, src in enumerate((rsw_h, rs2_h, rs3_h)):
            sl = sizes[t] // ns
            done = 0
            while done < sl:
                piece = min(chunk, sl - done)
                off = sid * sl + done
                pltpu.sync_copy(src.at[pl.ds(off, piece)],
                                obuf_v[0].at[pl.ds(0, piece)])
                pltpu.sync_copy(obuf_v[0].at[pl.ds(0, piece)],
                                rs_s[t].at[pl.ds(off, piece)])
                done += piece
        plsc.subcore_barrier()
        rs_h = (rs_s[0], rs_s[1], rs_s[2])
        ix_h = (iw_h, i2_h, i3_h)
        gathers = [[None] * 3 for _ in range(nchunk)]
        stores = [None] * nchunk
        for c in range(nchunk + 1):
            if c < nchunk:
                b = c % 2
                base = wid * per_w + c * chunk
                for t in range(3):
                    pltpu.sync_copy(ix_h[t].at[pl.ds(base, chunk)], idx_v[b][t])
                    gathers[c][t] = pltpu.async_copy(
                        rs_h[t].at[idx_v[b][t]], val_v[b][t], gsem[b][t])
            if c >= 1:
                d = c - 1
                pb = d % 2
                for t in range(3):
                    gathers[d][t].wait()
                if d >= 2:
                    stores[d - 2].wait()

                def add_vec(i, _, pb=pb):
                    s = pl.ds(i * 16, 16)
                    obuf_v[pb][s] = (val_v[pb][0][s] + val_v[pb][1][s]
                                     + val_v[pb][2][s])
                    return 0

                lax.fori_loop(0, chunk // 16, add_vec, 0, unroll=8)
                dbase = wid * per_w + d * chunk
                stores[d] = pltpu.async_copy(
                    obuf_v[pb], out_h.at[pl.ds(dbase, chunk)], osem[pb])
        stores[nchunk - 2].wait()
        stores[nchunk - 1].wait()

    return k(rs_w, rs_2, rs_3, idx_w, idx_2, idx_3)


def _mlp_head(x, w1, b1, w2, b2):
    bsz, l = x.shape
    d = w1.shape[1]
    nl = w2.shape[1]
    blk = 512

    def body(x_ref, w1_ref, b1_ref, w2_ref, b2_ref, o_ref):
        h = jnp.dot(x_ref[...], w1_ref[...], preferred_element_type=jnp.float32)
        h = jnp.maximum(h + b1_ref[...], 0.0)
        o_ref[...] = (
            jnp.dot(h, w2_ref[...], preferred_element_type=jnp.float32)
            + b2_ref[...]
        )

    return pl.pallas_call(
        body,
        grid=(bsz // blk,),
        in_specs=[
            pl.BlockSpec((blk, l), lambda i: (i, 0)),
            pl.BlockSpec((l, d), lambda i: (0, 0)),
            pl.BlockSpec((1, d), lambda i: (0, 0)),
            pl.BlockSpec((d, nl), lambda i: (0, 0)),
            pl.BlockSpec((1, nl), lambda i: (0, 0)),
        ],
        out_specs=pl.BlockSpec((blk, nl), lambda i: (i, 0)),
        out_shape=jax.ShapeDtypeStruct((bsz, nl), jnp.float32),
    )(x, w1, b1, w2, b2)


def kernel(input_ids, input_ids_gram2, input_ids_gram3, input_mask, labels,
           emb_word, emb_g2, emb_g3, W1, b1, W2, b2):
    bsz, l = input_ids.shape
    d = W1.shape[1]
    num_labels = W2.shape[1]
    scale = 1.0 / float(l)

    rs_w = _rowsums_scaled(emb_word, scale)
    rs_2 = _rowsums_scaled(emb_g2, scale)
    rs_3 = _rowsums_scaled(emb_g3, scale)

    pooled = _sc_gather_sum(
        rs_w, rs_2, rs_3,
        input_ids.reshape(-1),
        input_ids_gram2.reshape(-1),
        input_ids_gram3.reshape(-1),
    ).reshape(bsz, l)

    return _mlp_head(pooled, W1, b1.reshape(1, d), W2,
                     b2.reshape(1, num_labels))


# speedup vs baseline: 2.7968x; 1.0442x over previous
import functools

import jax
import jax.numpy as jnp
from jax import lax
from jax.experimental import pallas as pl
from jax.experimental.pallas import tpu as pltpu
from jax.experimental.pallas import tpu_sc as plsc


def _cdiv(a, b):
    return (a + b - 1) // b


def _rowsums_scaled(table, scale):
    n, d = table.shape
    blk = 32768
    nrows = _cdiv(n, 128)
    ngrid = _cdiv(n, blk)

    def body(t_ref, o_ref):
        x = t_ref[...].reshape(blk // 128, 128, d)
        o_ref[...] = jnp.sum(x, axis=2) * scale

    out = pl.pallas_call(
        body,
        grid=(ngrid,),
        in_specs=[pl.BlockSpec((blk, d), lambda i: (i, 0))],
        out_specs=pl.BlockSpec((blk // 128, 128), lambda i: (i, 0)),
        out_shape=jax.ShapeDtypeStruct((nrows, 128), jnp.float32),
    )(table)
    return out.reshape(-1)


def _sc_gather_sum(rs_w, rs_2, rs_3, idx_w, idx_2, idx_3):
    info = plsc.get_sparse_core_info()
    nw = info.num_cores * info.num_subcores
    ns = info.num_subcores
    ntot = idx_w.shape[0]
    per_w = ntot // nw
    chunk = 6144
    nchunk = per_w // chunk
    assert ntot % nw == 0 and per_w % chunk == 0
    sizes = (rs_w.shape[0], rs_2.shape[0], rs_3.shape[0])
    assert all(sz % (8 * ns) == 0 for sz in sizes)
    mesh = plsc.VectorSubcoreMesh(core_axis_name="c", subcore_axis_name="s")

    vmem_i = pltpu.VMEM((chunk,), jnp.int32)
    vmem_f = pltpu.VMEM((chunk,), jnp.float32)

    @functools.partial(
        pl.kernel,
        mesh=mesh,
        out_type=jax.ShapeDtypeStruct((ntot,), jnp.float32),
        scratch_types=[
            [pltpu.VMEM_SHARED((sz,), jnp.float32) for sz in sizes],
            [[vmem_i] * 3] * 2,
            [[vmem_f] * 3] * 2,
            [vmem_f] * 2,
            [[pltpu.SemaphoreType.DMA] * 3] * 2,
            [pltpu.SemaphoreType.DMA] * 2,
        ],
    )
    def k(rsw_h, rs2_h, rs3_h, iw_h, i2_h, i3_h, out_h,
          rs_s, idx_v, val_v, obuf_v, gsem, osem):
        wid = lax.axis_index("s") * info.num_cores + lax.axis_index("c")
        sid = lax.axis_index("s")
        for t, src in enumerate((rsw_h, rs2_h, rs3_h)):
            sl = sizes[t] // ns
            done = 0
            while done < sl:
                piece = min(chunk, sl - done)
                off = sid * sl + done
                pltpu.sync_copy(src.at[pl.ds(off, piece)],
                                obuf_v[0].at[pl.ds(0, piece)])
                pltpu.sync_copy(obuf_v[0].at[pl.ds(0, piece)],
                                rs_s[t].at[pl.ds(off, piece)])
                done += piece
        plsc.subcore_barrier()
        rs_h = (rs_s[0], rs_s[1], rs_s[2])
        ix_h = (iw_h, i2_h, i3_h)
        gathers = [[None] * 3 for _ in range(nchunk)]
        stores = [None] * nchunk
        for c in range(nchunk + 1):
            if c < nchunk:
                b = c % 2
                base = wid * per_w + c * chunk
                for t in range(3):
                    pltpu.sync_copy(ix_h[t].at[pl.ds(base, chunk)], idx_v[b][t])
                    gathers[c][t] = pltpu.async_copy(
                        rs_h[t].at[idx_v[b][t]], val_v[b][t], gsem[b][t])
            if c >= 1:
                d = c - 1
                pb = d % 2
                for t in range(3):
                    gathers[d][t].wait()
                if d >= 2:
                    stores[d - 2].wait()

                def add_vec(i, _, pb=pb):
                    s = pl.ds(i * 16, 16)
                    obuf_v[pb][s] = (val_v[pb][0][s] + val_v[pb][1][s]
                                     + val_v[pb][2][s])
                    return 0

                lax.fori_loop(0, chunk // 16, add_vec, 0, unroll=8)
                dbase = wid * per_w + d * chunk
                stores[d] = pltpu.async_copy(
                    obuf_v[pb], out_h.at[pl.ds(dbase, chunk)], osem[pb])
        stores[nchunk - 2].wait()
        stores[nchunk - 1].wait()

    return k(rs_w, rs_2, rs_3, idx_w, idx_2, idx_3)


def _mlp_head(x, w1, b1, w2, b2):
    bsz, l = x.shape
    d = w1.shape[1]
    nl = w2.shape[1]
    blk = 1024

    def body(x_ref, w1_ref, b1_ref, w2_ref, b2_ref, o_ref):
        h = jnp.dot(x_ref[...], w1_ref[...], preferred_element_type=jnp.float32)
        h = jnp.maximum(h + b1_ref[...], 0.0)
        o_ref[...] = (
            jnp.dot(h, w2_ref[...], preferred_element_type=jnp.float32)
            + b2_ref[...]
        )

    return pl.pallas_call(
        body,
        grid=(bsz // blk,),
        in_specs=[
            pl.BlockSpec((blk, l), lambda i: (i, 0)),
            pl.BlockSpec((l, d), lambda i: (0, 0)),
            pl.BlockSpec((1, d), lambda i: (0, 0)),
            pl.BlockSpec((d, nl), lambda i: (0, 0)),
            pl.BlockSpec((1, nl), lambda i: (0, 0)),
        ],
        out_specs=pl.BlockSpec((blk, nl), lambda i: (i, 0)),
        out_shape=jax.ShapeDtypeStruct((bsz, nl), jnp.float32),
    )(x, w1, b1, w2, b2)


def kernel(input_ids, input_ids_gram2, input_ids_gram3, input_mask, labels,
           emb_word, emb_g2, emb_g3, W1, b1, W2, b2):
    bsz, l = input_ids.shape
    d = W1.shape[1]
    num_labels = W2.shape[1]
    scale = 1.0 / float(l)

    rs_w = _rowsums_scaled(emb_word, scale)
    rs_2 = _rowsums_scaled(emb_g2, scale)
    rs_3 = _rowsums_scaled(emb_g3, scale)

    pooled = _sc_gather_sum(
        rs_w, rs_2, rs_3,
        input_ids.reshape(-1),
        input_ids_gram2.reshape(-1),
        input_ids_gram3.reshape(-1),
    ).reshape(bsz, l)

    return _mlp_head(pooled, W1, b1.reshape(1, d), W2,
                     b2.reshape(1, num_labels))
